# baseline (speedup 1.0000x reference)
import functools

import jax
import jax.numpy as jnp
from jax.experimental import pallas as pl

_NUM_EMBEDDINGS = 1024
_EMBEDDING_DIM = 64
_COMMITMENT_COST = 0.25


def _vq_block(x_ref, w_ref, idx_ref, qst_ref, loss_ref):
    x = x_ref[0]
    w = w_ref[...]
    f = x.T
    fs = jnp.sum(f * f, axis=1, keepdims=True)
    s = jax.lax.dot_general(
        f, w, (((1,), (1,)), ((), ())),
        preferred_element_type=jnp.float32)
    ws = jnp.sum(w * w, axis=1)[None, :]
    d = fs - 2.0 * s + ws
    minval = jnp.min(d, axis=1, keepdims=True)
    jidx = jax.lax.broadcasted_iota(jnp.int32, d.shape, 1).astype(jnp.float32)
    idx_f = jnp.min(jnp.where(d == minval, jidx, 2048.0),
                    axis=1, keepdims=True)
    idx_ref[...] = idx_f.astype(jnp.int32)
    onehot = (jidx == idx_f).astype(jnp.bfloat16)
    q = jax.lax.dot_general(
        w.astype(jnp.bfloat16), onehot, (((0,), (1,)), ((), ())),
        preferred_element_type=jnp.float32)
    qst_ref[0] = x + (q - x)
    part = jnp.sum((q - x) ** 2)
    @pl.when(pl.program_id(0) == 0)
    def _init():
        loss_ref[...] = jnp.zeros_like(loss_ref)
    loss_ref[...] += part[None, None]


@functools.partial(jax.jit, static_argnames=())
def kernel(inputs, W):
    b, c, h, w = inputs.shape
    hw = h * w
    n = b * hw
    x3 = inputs.reshape(b, c, hw)
    idx2, qst3, loss_sum = pl.pallas_call(
        _vq_block,
        grid=(b,),
        in_specs=[
            pl.BlockSpec((1, c, hw), lambda i: (i, 0, 0)),
            pl.BlockSpec((_NUM_EMBEDDINGS, c), lambda i: (0, 0)),
        ],
        out_specs=[
            pl.BlockSpec((hw, 1), lambda i: (i, 0)),
            pl.BlockSpec((1, c, hw), lambda i: (i, 0, 0)),
            pl.BlockSpec((1, 1), lambda i: (0, 0)),
        ],
        out_shape=[
            jax.ShapeDtypeStruct((n, 1), jnp.int32),
            jax.ShapeDtypeStruct((b, c, hw), jnp.float32),
            jax.ShapeDtypeStruct((1, 1), jnp.float32),
        ],
    )(x3, W)
    discrete = idx2.reshape(b, h, w)
    quantized_out = qst3.reshape(b, c, h, w)
    m = loss_sum[0, 0] / n / c
    loss = m + _COMMITMENT_COST * m
    return (discrete, quantized_out, loss)


# speedup vs baseline: 1.0769x; 1.0769x over previous
import functools

import jax
import jax.numpy as jnp
from jax.experimental import pallas as pl

_NUM_EMBEDDINGS = 1024
_EMBEDDING_DIM = 64
_COMMITMENT_COST = 0.25


def _vq_block(x_ref, w_ref, idx_ref, qst_ref, loss_ref):
    nb = x_ref.shape[0]
    w = w_ref[...]
    f = jnp.concatenate([x_ref[i].T for i in range(nb)],
                        axis=0)
    fs = jnp.sum(f * f, axis=1, keepdims=True)
    s = jax.lax.dot_general(
        f, w, (((1,), (1,)), ((), ())),
        preferred_element_type=jnp.float32)
    ws = jnp.sum(w * w, axis=1)[None, :]
    d = fs - 2.0 * s + ws
    minval = jnp.min(d, axis=1, keepdims=True)
    jidx = jax.lax.broadcasted_iota(jnp.int32, d.shape, 1).astype(jnp.float32)
    idx_f = jnp.min(jnp.where(d == minval, jidx, 2048.0),
                    axis=1, keepdims=True)
    idx_ref[...] = idx_f.astype(jnp.int32)
    onehot = (jidx == idx_f).astype(jnp.bfloat16)
    q = jax.lax.dot_general(
        w.astype(jnp.bfloat16), onehot, (((0,), (1,)), ((), ())),
        preferred_element_type=jnp.float32)
    hw = q.shape[1] // nb
    part = jnp.float32(0.0)
    for i in range(nb):
        xi = x_ref[i]
        qi = q[:, i * hw:(i + 1) * hw]
        qst_ref[i] = xi + (qi - xi)
        part += jnp.sum((qi - xi) ** 2)
    @pl.when(pl.program_id(0) == 0)
    def _init():
        loss_ref[...] = jnp.zeros_like(loss_ref)
    loss_ref[...] += part[None, None]


@functools.partial(jax.jit, static_argnames=())
def kernel(inputs, W):
    b, c, h, w = inputs.shape
    hw = h * w
    n = b * hw
    x3 = inputs.reshape(b, c, hw)
    nb = 2
    idx2, qst3, loss_sum = pl.pallas_call(
        _vq_block,
        grid=(b // nb,),
        in_specs=[
            pl.BlockSpec((nb, c, hw), lambda i: (i, 0, 0)),
            pl.BlockSpec((_NUM_EMBEDDINGS, c), lambda i: (0, 0)),
        ],
        out_specs=[
            pl.BlockSpec((nb * hw, 1), lambda i: (i, 0)),
            pl.BlockSpec((nb, c, hw), lambda i: (i, 0, 0)),
            pl.BlockSpec((1, 1), lambda i: (0, 0)),
        ],
        out_shape=[
            jax.ShapeDtypeStruct((n, 1), jnp.int32),
            jax.ShapeDtypeStruct((b, c, hw), jnp.float32),
            jax.ShapeDtypeStruct((1, 1), jnp.float32),
        ],
    )(x3, W)
    discrete = idx2.reshape(b, h, w)
    quantized_out = qst3.reshape(b, c, h, w)
    m = loss_sum[0, 0] / n / c
    loss = m + _COMMITMENT_COST * m
    return (discrete, quantized_out, loss)
